# baseline (speedup 1.0000x reference)
import functools

import jax
import jax.numpy as jnp
from jax import lax
from jax.experimental import pallas as pl
from jax.experimental.pallas import tpu as pltpu
from jax.experimental.pallas import tpu_sc as plsc

N = 10000
E = 320000
IN_DIM = 128
HID = 256
NCLS = 64

NC, NS = 2, 16
NW = NC * NS
K = 128
PAD_E = 327680
NSLOT = 2
N_SINK = 16
NP = N + N_SINK
TPT = 624
TAIL_I = NP - TPT * NS
TAIL_W = N - TPT * NS


@functools.lru_cache(maxsize=None)
def _make_agg(dc, feat_split):
    epw = PAD_E // NS if feat_split else PAD_E // NW
    nch = epw // K
    SCH = NSLOT
    nsc = nch // SCH
    mesh = plsc.VectorSubcoreMesh(core_axis_name="c", subcore_axis_name="s",
                                  num_cores=NC, num_subcores=NS)

    def body(x_hbm, src_hbm, dst_hbm, zero_hbm, out_hbm,
             sidx, didx, rows, acc, gsem0, gsem1, ssem0, ssem1,
             isem, zsem):
        gsem = {0: gsem0, 1: gsem1}
        ssem = {0: ssem0, 1: ssem1}
        c = lax.axis_index("c")
        s = lax.axis_index("s")
        crow = (s if feat_split else c * NS + s) * nch
        xref = x_hbm.at[c] if feat_split else x_hbm

        def idx_load(scn, p):
            r = crow + scn * SCH
            pltpu.async_copy(src_hbm.at[pl.ds(r, SCH)], sidx.at[p], isem)
            pltpu.async_copy(dst_hbm.at[pl.ds(r, SCH)], didx.at[p], isem)

        def idx_wait(scn, p):
            r = crow + scn * SCH
            pltpu.make_async_copy(src_hbm.at[pl.ds(r, SCH)], sidx.at[p],
                                  isem).wait()
            pltpu.make_async_copy(dst_hbm.at[pl.ds(r, SCH)], didx.at[p],
                                  isem).wait()

        def start_g(p, k, b):
            pltpu.async_copy(xref.at[sidx.at[p, k]], rows.at[b],
                             gsem[b % 2])

        def wait_g(p, k, b):
            pltpu.make_async_copy(xref.at[sidx.at[p, k]], rows.at[b],
                                  gsem[b % 2]).wait()

        def start_s(p, k, b):
            pltpu.async_copy(rows.at[b], acc.at[didx.at[p, k]],
                             ssem[b % 2], add=True)

        def wait_s(p, k, b):
            pltpu.make_async_copy(rows.at[b], acc.at[didx.at[p, k]],
                                  ssem[b % 2]).wait()

        ini = pltpu.async_copy(zero_hbm.at[pl.ds(s * TPT, TPT)],
                               acc.at[pl.ds(s * TPT, TPT)], zsem)

        @pl.when(s == 0)
        def _init_tail():
            pltpu.async_copy(zero_hbm.at[pl.ds(TPT * NS, TAIL_I)],
                             acc.at[pl.ds(TPT * NS, TAIL_I)], zsem)

        pltpu.sync_copy(src_hbm.at[pl.ds(crow, SCH)], sidx.at[0])
        pltpu.sync_copy(dst_hbm.at[pl.ds(crow, SCH)], didx.at[0])
        idx_load(1, 1)
        ini.wait()

        @pl.when(s == 0)
        def _init_tail_wait():
            pltpu.make_async_copy(zero_hbm.at[pl.ds(TPT * NS, TAIL_I)],
                                  acc.at[pl.ds(TPT * NS, TAIL_I)],
                                  zsem).wait()

        plsc.subcore_barrier()

        start_g(0, 0, 0)
        start_g(0, 1, 1)

        def sc_body(scn, carry):
            p = scn % 2
            for k in range(SCH):
                b = k % 2
                wait_g(p, k, b)
                start_s(p, k, b)
                wait_s(p, k, b)
                if k < SCH - 2:
                    start_g(p, k + 2, b)
                elif k == SCH - 2:
                    @pl.when(scn + 1 < nsc)
                    def _pf0():
                        idx_wait(scn + 1, 1 - p)
                        start_g(1 - p, 0, 0)
                else:
                    @pl.when(scn + 1 < nsc)
                    def _pf1():
                        start_g(1 - p, 1, 1)
            @pl.when(scn + 2 < nsc)
            def _pfi():
                idx_load(scn + 2, p)
            return carry

        lax.fori_loop(0, nsc, sc_body, 0)

        plsc.subcore_barrier()
        pltpu.sync_copy(acc.at[pl.ds(s * TPT, TPT)],
                        out_hbm.at[c, pl.ds(s * TPT, TPT)])

        @pl.when(s == 0)
        def _write_tail():
            pltpu.sync_copy(acc.at[pl.ds(TPT * NS, TAIL_W)],
                            out_hbm.at[c, pl.ds(TPT * NS, TAIL_W)])

    return pl.kernel(
        body,
        out_type=jax.ShapeDtypeStruct((NC, N, dc), jnp.float32),
        mesh=mesh,
        scratch_types=[
            pltpu.VMEM((2, SCH, K), jnp.int32),
            pltpu.VMEM((2, SCH, K), jnp.int32),
            pltpu.VMEM((NSLOT, K, dc), jnp.float32),
            pltpu.VMEM_SHARED((NP, dc), jnp.float32),
            pltpu.SemaphoreType.DMA,
            pltpu.SemaphoreType.DMA,
            pltpu.SemaphoreType.DMA,
            pltpu.SemaphoreType.DMA,
            pltpu.SemaphoreType.DMA,
            pltpu.SemaphoreType.DMA,
        ],
    )


@functools.lru_cache(maxsize=None)
def _make_deg():
    nch = PAD_E // NW // K
    SCH = NSLOT
    nsc = nch // SCH
    mesh = plsc.VectorSubcoreMesh(core_axis_name="c", subcore_axis_name="s",
                                  num_cores=NC, num_subcores=NS)

    def body(ones_hbm, dst_hbm, zero_hbm, out_hbm,
             didx, rows, acc, ssem0, ssem1, isem, zsem):
        ssem = {0: ssem0, 1: ssem1}
        c = lax.axis_index("c")
        s = lax.axis_index("s")
        crow = (c * NS + s) * nch

        def idx_load(scn, p):
            pltpu.async_copy(dst_hbm.at[pl.ds(crow + scn * SCH, SCH)],
                             didx.at[p], isem)

        def idx_wait(scn, p):
            pltpu.make_async_copy(dst_hbm.at[pl.ds(crow + scn * SCH, SCH)],
                                  didx.at[p], isem).wait()

        def start_s(p, k, b):
            pltpu.async_copy(rows, acc.at[didx.at[p, k]], ssem[b % 2],
                             add=True)

        def wait_s(p, k, b):
            pltpu.make_async_copy(rows, acc.at[didx.at[p, k]],
                                  ssem[b % 2]).wait()

        ini = pltpu.async_copy(zero_hbm.at[pl.ds(s * TPT, TPT)],
                               acc.at[pl.ds(s * TPT, TPT)], zsem)

        @pl.when(s == 0)
        def _init_tail():
            pltpu.async_copy(zero_hbm.at[pl.ds(TPT * NS, TAIL_I)],
                             acc.at[pl.ds(TPT * NS, TAIL_I)], zsem)

        pltpu.sync_copy(dst_hbm.at[pl.ds(crow, SCH)], didx.at[0])
        idx_load(1, 1)
        pltpu.sync_copy(ones_hbm, rows)
        ini.wait()

        @pl.when(s == 0)
        def _init_tail_wait():
            pltpu.make_async_copy(zero_hbm.at[pl.ds(TPT * NS, TAIL_I)],
                                  acc.at[pl.ds(TPT * NS, TAIL_I)],
                                  zsem).wait()

        plsc.subcore_barrier()

        def sc_body(scn, carry):
            p = scn % 2
            for k in range(SCH):
                start_s(p, k, k)
                if k > 0:
                    wait_s(p, k - 1, k - 1)
                if k == SCH - 2:
                    @pl.when(scn + 1 < nsc)
                    def _pf():
                        idx_wait(scn + 1, 1 - p)
            wait_s(p, SCH - 1, SCH - 1)
            @pl.when(scn + 2 < nsc)
            def _pfi():
                idx_load(scn + 2, p)
            return carry

        lax.fori_loop(0, nsc, sc_body, 0)

        plsc.subcore_barrier()
        pltpu.sync_copy(acc.at[pl.ds(s * TPT, TPT)],
                        out_hbm.at[c, pl.ds(s * TPT, TPT)])

        @pl.when(s == 0)
        def _write_tail():
            pltpu.sync_copy(acc.at[pl.ds(TPT * NS, TAIL_W)],
                            out_hbm.at[c, pl.ds(TPT * NS, TAIL_W)])

    return pl.kernel(
        body,
        out_type=jax.ShapeDtypeStruct((NC, N, IN_DIM), jnp.float32),
        mesh=mesh,
        scratch_types=[
            pltpu.VMEM((2, SCH, K), jnp.int32),
            pltpu.VMEM((K, IN_DIM), jnp.float32),
            pltpu.VMEM_SHARED((NP, IN_DIM), jnp.float32),
            pltpu.SemaphoreType.DMA,
            pltpu.SemaphoreType.DMA,
            pltpu.SemaphoreType.DMA,
            pltpu.SemaphoreType.DMA,
        ],
    )


_RB = 1000


def _tc_prep(degp, x):
    def body(dp, xr, dis_o, x0_o):
        deg = dp[0, :, 0:1] + dp[1, :, 0:1] + 1.0
        d = lax.rsqrt(deg)
        dis_o[...] = d
        x0_o[...] = xr[...] * d

    return pl.pallas_call(
        body,
        grid=(N // _RB,),
        in_specs=[pl.BlockSpec((NC, _RB, IN_DIM), lambda i: (0, i, 0)),
                  pl.BlockSpec((_RB, IN_DIM), lambda i: (i, 0))],
        out_specs=[pl.BlockSpec((_RB, 1), lambda i: (i, 0)),
                   pl.BlockSpec((_RB, IN_DIM), lambda i: (i, 0))],
        out_shape=[jax.ShapeDtypeStruct((N, 1), jnp.float32),
                   jax.ShapeDtypeStruct((N, IN_DIM), jnp.float32)],
    )(degp, x)


def _tc_layer1(y1p, x0, dis, w1, b1, w2):
    def body(yp, x0r, dr, w1r, b1r, w2r, out):
        z = (yp[0] + yp[1] + x0r[...]) * dr[...]
        h = jnp.maximum(
            jnp.dot(z, w1r[...], preferred_element_type=jnp.float32) + b1r[...],
            0.0)
        t = jnp.dot(h, w2r[...], preferred_element_type=jnp.float32) * dr[...]
        out[0, :, :] = t[:, :IN_DIM]
        out[1, :, :] = t[:, IN_DIM:]

    return pl.pallas_call(
        body,
        grid=(N // _RB,),
        in_specs=[pl.BlockSpec((NC, _RB, IN_DIM), lambda i: (0, i, 0)),
                  pl.BlockSpec((_RB, IN_DIM), lambda i: (i, 0)),
                  pl.BlockSpec((_RB, 1), lambda i: (i, 0)),
                  pl.BlockSpec((IN_DIM, HID), lambda i: (0, 0)),
                  pl.BlockSpec((1, HID), lambda i: (0, 0)),
                  pl.BlockSpec((HID, HID), lambda i: (0, 0))],
        out_specs=pl.BlockSpec((NC, _RB, IN_DIM), lambda i: (0, i, 0)),
        out_shape=jax.ShapeDtypeStruct((NC, N, IN_DIM), jnp.float32),
    )(y1p, x0, dis, w1, b1, w2)


def _tc_layer2(y2s, t2s, dis, b2, w3):
    def body(yr, tr, dr, b2r, w3r, out):
        u0 = (yr[0] + tr[0]) * dr[...]
        u1 = (yr[1] + tr[1]) * dr[...]
        h = jnp.maximum(jnp.concatenate([u0, u1], axis=1) + b2r[...], 0.0)
        out[...] = jnp.dot(h, w3r[...],
                           preferred_element_type=jnp.float32) * dr[...]

    return pl.pallas_call(
        body,
        grid=(N // _RB,),
        in_specs=[pl.BlockSpec((NC, _RB, IN_DIM), lambda i: (0, i, 0)),
                  pl.BlockSpec((NC, _RB, IN_DIM), lambda i: (0, i, 0)),
                  pl.BlockSpec((_RB, 1), lambda i: (i, 0)),
                  pl.BlockSpec((1, HID), lambda i: (0, 0)),
                  pl.BlockSpec((HID, IN_DIM), lambda i: (0, 0))],
        out_specs=pl.BlockSpec((_RB, IN_DIM), lambda i: (i, 0)),
        out_shape=jax.ShapeDtypeStruct((N, IN_DIM), jnp.float32),
    )(y2s, t2s, dis, b2, w3)


def _tc_final(y3p, t3, dis, b3):
    def body(yp, tr, dr, b3r, out):
        out[...] = ((yp[0] + yp[1] + tr[...]) * dr[...])[:, :NCLS] + b3r[...]

    return pl.pallas_call(
        body,
        grid=(N // _RB,),
        in_specs=[pl.BlockSpec((NC, _RB, IN_DIM), lambda i: (0, i, 0)),
                  pl.BlockSpec((_RB, IN_DIM), lambda i: (i, 0)),
                  pl.BlockSpec((_RB, 1), lambda i: (i, 0)),
                  pl.BlockSpec((1, NCLS), lambda i: (0, 0))],
        out_specs=pl.BlockSpec((_RB, NCLS), lambda i: (i, 0)),
        out_shape=jax.ShapeDtypeStruct((N, NCLS), jnp.float32),
    )(y3p, t3, dis, b3)


def kernel(X, A, W1, b1, W2, b2, W3, b3):
    src = A[0].astype(jnp.int32)
    dst = A[1].astype(jnp.int32)
    padi = jnp.arange(PAD_E - E, dtype=jnp.int32)
    srcp = jnp.concatenate([src, padi % 8]).reshape(PAD_E // K, K)
    dstp = jnp.concatenate([dst, N + (padi % N_SINK)]).reshape(PAD_E // K, K)
    ones_k = jnp.ones((K, IN_DIM), jnp.float32)
    z128 = jnp.zeros((NP, IN_DIM), jnp.float32)
    w3p = jnp.pad(W3, ((0, 0), (0, IN_DIM - NCLS)))

    degp = _make_deg()(ones_k, dstp, z128)
    dis, x0 = _tc_prep(degp, X)
    y1p = _make_agg(128, False)(x0, srcp, dstp, z128)
    t2s = _tc_layer1(y1p, x0, dis, W1, b1.reshape(1, HID), W2)
    y2s = _make_agg(128, True)(t2s, srcp, dstp, z128)
    t3 = _tc_layer2(y2s, t2s, dis, b2.reshape(1, HID), w3p)
    y3p = _make_agg(128, False)(t3, srcp, dstp, z128)
    return _tc_final(y3p, t3, dis, b3.reshape(1, NCLS))


# speedup vs baseline: 1.1262x; 1.1262x over previous
import functools

import jax
import jax.numpy as jnp
from jax import lax
from jax.experimental import pallas as pl
from jax.experimental.pallas import tpu as pltpu
from jax.experimental.pallas import tpu_sc as plsc

N = 10000
E = 320000
IN_DIM = 128
HID = 256
NCLS = 64

NC, NS = 2, 16
NW = NC * NS
K = 80
PAD_E = 327680
NSLOT = 4
N_SINK = 16
NP = N + N_SINK
TPT = 624
TAIL_I = NP - TPT * NS
TAIL_W = N - TPT * NS


@functools.lru_cache(maxsize=None)
def _make_agg(dc, feat_split):
    epw = PAD_E // NS if feat_split else PAD_E // NW
    nch = epw // K
    SCH = NSLOT
    nsc = nch // SCH
    mesh = plsc.VectorSubcoreMesh(core_axis_name="c", subcore_axis_name="s",
                                  num_cores=NC, num_subcores=NS)

    def body(x_hbm, src_hbm, dst_hbm, zero_hbm, out_hbm,
             sidx, didx, rows, acc, gsem0, gsem1, gsem2, gsem3,
             ssem0, ssem1, ssem2, ssem3, isem, zsem):
        gsem = {0: gsem0, 1: gsem1, 2: gsem2, 3: gsem3}
        ssem = {0: ssem0, 1: ssem1, 2: ssem2, 3: ssem3}
        c = lax.axis_index("c")
        s = lax.axis_index("s")
        crow = (s if feat_split else c * NS + s) * nch
        xref = x_hbm.at[c] if feat_split else x_hbm

        def idx_load(scn, p):
            r = crow + scn * SCH
            pltpu.async_copy(src_hbm.at[pl.ds(r, SCH)], sidx.at[p], isem)
            pltpu.async_copy(dst_hbm.at[pl.ds(r, SCH)], didx.at[p], isem)

        def idx_wait(scn, p):
            r = crow + scn * SCH
            pltpu.make_async_copy(src_hbm.at[pl.ds(r, SCH)], sidx.at[p],
                                  isem).wait()
            pltpu.make_async_copy(dst_hbm.at[pl.ds(r, SCH)], didx.at[p],
                                  isem).wait()

        def start_g(p, k, b):
            pltpu.async_copy(xref.at[sidx.at[p, k]], rows.at[b], gsem[b])

        def wait_g(p, k, b):
            pltpu.make_async_copy(xref.at[sidx.at[p, k]], rows.at[b],
                                  gsem[b]).wait()

        def start_s(p, k, b):
            pltpu.async_copy(rows.at[b], acc.at[didx.at[p, k]],
                             ssem[b], add=True)

        def wait_s(p, k, b):
            pltpu.make_async_copy(rows.at[b], acc.at[didx.at[p, k]],
                                  ssem[b]).wait()

        ini = pltpu.async_copy(zero_hbm.at[pl.ds(s * TPT, TPT)],
                               acc.at[pl.ds(s * TPT, TPT)], zsem)

        @pl.when(s == 0)
        def _init_tail():
            pltpu.async_copy(zero_hbm.at[pl.ds(TPT * NS, TAIL_I)],
                             acc.at[pl.ds(TPT * NS, TAIL_I)], zsem)

        pltpu.sync_copy(src_hbm.at[pl.ds(crow, SCH)], sidx.at[0])
        pltpu.sync_copy(dst_hbm.at[pl.ds(crow, SCH)], didx.at[0])
        idx_load(1, 1)
        ini.wait()

        @pl.when(s == 0)
        def _init_tail_wait():
            pltpu.make_async_copy(zero_hbm.at[pl.ds(TPT * NS, TAIL_I)],
                                  acc.at[pl.ds(TPT * NS, TAIL_I)],
                                  zsem).wait()

        plsc.subcore_barrier()

        start_g(0, 0, 0)
        start_g(0, 1, 1)
        start_g(0, 2, 2)

        def sc_body(scn, carry):
            p = scn % 2
            for k in range(SCH):
                wait_g(p, k, k)
                start_s(p, k, k)
                if k > 0:
                    wait_s(p, k - 1, k - 1)
                if k == 0:
                    start_g(p, 3, 3)
                elif k == 1:
                    @pl.when(scn + 1 < nsc)
                    def _pf0():
                        idx_wait(scn + 1, 1 - p)
                        start_g(1 - p, 0, 0)
                elif k == 2:
                    @pl.when(scn + 1 < nsc)
                    def _pf1():
                        start_g(1 - p, 1, 1)
                else:
                    @pl.when(scn + 1 < nsc)
                    def _pf2():
                        start_g(1 - p, 2, 2)
            wait_s(p, SCH - 1, SCH - 1)
            @pl.when(scn + 2 < nsc)
            def _pfi():
                idx_load(scn + 2, p)
            return carry

        lax.fori_loop(0, nsc, sc_body, 0)

        plsc.subcore_barrier()
        pltpu.sync_copy(acc.at[pl.ds(s * TPT, TPT)],
                        out_hbm.at[c, pl.ds(s * TPT, TPT)])

        @pl.when(s == 0)
        def _write_tail():
            pltpu.sync_copy(acc.at[pl.ds(TPT * NS, TAIL_W)],
                            out_hbm.at[c, pl.ds(TPT * NS, TAIL_W)])

    return pl.kernel(
        body,
        out_type=jax.ShapeDtypeStruct((NC, N, dc), jnp.float32),
        mesh=mesh,
        scratch_types=[
            pltpu.VMEM((2, SCH, K), jnp.int32),
            pltpu.VMEM((2, SCH, K), jnp.int32),
            pltpu.VMEM((NSLOT, K, dc), jnp.float32),
            pltpu.VMEM_SHARED((NP, dc), jnp.float32),
            pltpu.SemaphoreType.DMA,
            pltpu.SemaphoreType.DMA,
            pltpu.SemaphoreType.DMA,
            pltpu.SemaphoreType.DMA,
            pltpu.SemaphoreType.DMA,
            pltpu.SemaphoreType.DMA,
            pltpu.SemaphoreType.DMA,
            pltpu.SemaphoreType.DMA,
            pltpu.SemaphoreType.DMA,
            pltpu.SemaphoreType.DMA,
        ],
    )


@functools.lru_cache(maxsize=None)
def _make_deg():
    nch = PAD_E // NW // K
    SCH = NSLOT
    nsc = nch // SCH
    mesh = plsc.VectorSubcoreMesh(core_axis_name="c", subcore_axis_name="s",
                                  num_cores=NC, num_subcores=NS)

    def body(ones_hbm, dst_hbm, zero_hbm, out_hbm,
             didx, rows, acc, ssem0, ssem1, isem, zsem):
        ssem = {0: ssem0, 1: ssem1}
        c = lax.axis_index("c")
        s = lax.axis_index("s")
        crow = (c * NS + s) * nch

        def idx_load(scn, p):
            pltpu.async_copy(dst_hbm.at[pl.ds(crow + scn * SCH, SCH)],
                             didx.at[p], isem)

        def idx_wait(scn, p):
            pltpu.make_async_copy(dst_hbm.at[pl.ds(crow + scn * SCH, SCH)],
                                  didx.at[p], isem).wait()

        def start_s(p, k, b):
            pltpu.async_copy(rows, acc.at[didx.at[p, k]], ssem[b % 2],
                             add=True)

        def wait_s(p, k, b):
            pltpu.make_async_copy(rows, acc.at[didx.at[p, k]],
                                  ssem[b % 2]).wait()

        ini = pltpu.async_copy(zero_hbm.at[pl.ds(s * TPT, TPT)],
                               acc.at[pl.ds(s * TPT, TPT)], zsem)

        @pl.when(s == 0)
        def _init_tail():
            pltpu.async_copy(zero_hbm.at[pl.ds(TPT * NS, TAIL_I)],
                             acc.at[pl.ds(TPT * NS, TAIL_I)], zsem)

        pltpu.sync_copy(dst_hbm.at[pl.ds(crow, SCH)], didx.at[0])
        idx_load(1, 1)
        pltpu.sync_copy(ones_hbm, rows)
        ini.wait()

        @pl.when(s == 0)
        def _init_tail_wait():
            pltpu.make_async_copy(zero_hbm.at[pl.ds(TPT * NS, TAIL_I)],
                                  acc.at[pl.ds(TPT * NS, TAIL_I)],
                                  zsem).wait()

        plsc.subcore_barrier()

        def sc_body(scn, carry):
            p = scn % 2
            for k in range(SCH):
                start_s(p, k, k)
                if k > 0:
                    wait_s(p, k - 1, k - 1)
                if k == SCH - 2:
                    @pl.when(scn + 1 < nsc)
                    def _pf():
                        idx_wait(scn + 1, 1 - p)
            wait_s(p, SCH - 1, SCH - 1)
            @pl.when(scn + 2 < nsc)
            def _pfi():
                idx_load(scn + 2, p)
            return carry

        lax.fori_loop(0, nsc, sc_body, 0)

        plsc.subcore_barrier()
        pltpu.sync_copy(acc.at[pl.ds(s * TPT, TPT)],
                        out_hbm.at[c, pl.ds(s * TPT, TPT)])

        @pl.when(s == 0)
        def _write_tail():
            pltpu.sync_copy(acc.at[pl.ds(TPT * NS, TAIL_W)],
                            out_hbm.at[c, pl.ds(TPT * NS, TAIL_W)])

    return pl.kernel(
        body,
        out_type=jax.ShapeDtypeStruct((NC, N, IN_DIM), jnp.float32),
        mesh=mesh,
        scratch_types=[
            pltpu.VMEM((2, SCH, K), jnp.int32),
            pltpu.VMEM((K, IN_DIM), jnp.float32),
            pltpu.VMEM_SHARED((NP, IN_DIM), jnp.float32),
            pltpu.SemaphoreType.DMA,
            pltpu.SemaphoreType.DMA,
            pltpu.SemaphoreType.DMA,
            pltpu.SemaphoreType.DMA,
        ],
    )


_RB = 1000


def _tc_prep(degp, x):
    def body(dp, xr, dis_o, x0_o):
        deg = dp[0, :, 0:1] + dp[1, :, 0:1] + 1.0
        d = lax.rsqrt(deg)
        dis_o[...] = d
        x0_o[...] = xr[...] * d

    return pl.pallas_call(
        body,
        grid=(N // _RB,),
        in_specs=[pl.BlockSpec((NC, _RB, IN_DIM), lambda i: (0, i, 0)),
                  pl.BlockSpec((_RB, IN_DIM), lambda i: (i, 0))],
        out_specs=[pl.BlockSpec((_RB, 1), lambda i: (i, 0)),
                   pl.BlockSpec((_RB, IN_DIM), lambda i: (i, 0))],
        out_shape=[jax.ShapeDtypeStruct((N, 1), jnp.float32),
                   jax.ShapeDtypeStruct((N, IN_DIM), jnp.float32)],
    )(degp, x)


def _tc_layer1(y1p, x0, dis, w1, b1, w2):
    def body(yp, x0r, dr, w1r, b1r, w2r, out):
        z = (yp[0] + yp[1] + x0r[...]) * dr[...]
        h = jnp.maximum(
            jnp.dot(z, w1r[...], preferred_element_type=jnp.float32) + b1r[...],
            0.0)
        t = jnp.dot(h, w2r[...], preferred_element_type=jnp.float32) * dr[...]
        out[0, :, :] = t[:, :IN_DIM]
        out[1, :, :] = t[:, IN_DIM:]

    return pl.pallas_call(
        body,
        grid=(N // _RB,),
        in_specs=[pl.BlockSpec((NC, _RB, IN_DIM), lambda i: (0, i, 0)),
                  pl.BlockSpec((_RB, IN_DIM), lambda i: (i, 0)),
                  pl.BlockSpec((_RB, 1), lambda i: (i, 0)),
                  pl.BlockSpec((IN_DIM, HID), lambda i: (0, 0)),
                  pl.BlockSpec((1, HID), lambda i: (0, 0)),
                  pl.BlockSpec((HID, HID), lambda i: (0, 0))],
        out_specs=pl.BlockSpec((NC, _RB, IN_DIM), lambda i: (0, i, 0)),
        out_shape=jax.ShapeDtypeStruct((NC, N, IN_DIM), jnp.float32),
    )(y1p, x0, dis, w1, b1, w2)


def _tc_layer2(y2s, t2s, dis, b2, w3):
    def body(yr, tr, dr, b2r, w3r, out):
        u0 = (yr[0] + tr[0]) * dr[...]
        u1 = (yr[1] + tr[1]) * dr[...]
        h = jnp.maximum(jnp.concatenate([u0, u1], axis=1) + b2r[...], 0.0)
        out[...] = jnp.dot(h, w3r[...],
                           preferred_element_type=jnp.float32) * dr[...]

    return pl.pallas_call(
        body,
        grid=(N // _RB,),
        in_specs=[pl.BlockSpec((NC, _RB, IN_DIM), lambda i: (0, i, 0)),
                  pl.BlockSpec((NC, _RB, IN_DIM), lambda i: (0, i, 0)),
                  pl.BlockSpec((_RB, 1), lambda i: (i, 0)),
                  pl.BlockSpec((1, HID), lambda i: (0, 0)),
                  pl.BlockSpec((HID, IN_DIM), lambda i: (0, 0))],
        out_specs=pl.BlockSpec((_RB, IN_DIM), lambda i: (i, 0)),
        out_shape=jax.ShapeDtypeStruct((N, IN_DIM), jnp.float32),
    )(y2s, t2s, dis, b2, w3)


def _tc_final(y3p, t3, dis, b3):
    def body(yp, tr, dr, b3r, out):
        out[...] = ((yp[0] + yp[1] + tr[...]) * dr[...])[:, :NCLS] + b3r[...]

    return pl.pallas_call(
        body,
        grid=(N // _RB,),
        in_specs=[pl.BlockSpec((NC, _RB, IN_DIM), lambda i: (0, i, 0)),
                  pl.BlockSpec((_RB, IN_DIM), lambda i: (i, 0)),
                  pl.BlockSpec((_RB, 1), lambda i: (i, 0)),
                  pl.BlockSpec((1, NCLS), lambda i: (0, 0))],
        out_specs=pl.BlockSpec((_RB, NCLS), lambda i: (i, 0)),
        out_shape=jax.ShapeDtypeStruct((N, NCLS), jnp.float32),
    )(y3p, t3, dis, b3)


def kernel(X, A, W1, b1, W2, b2, W3, b3):
    src = A[0].astype(jnp.int32)
    dst = A[1].astype(jnp.int32)
    padi = jnp.arange(PAD_E - E, dtype=jnp.int32)
    srcp = jnp.concatenate([src, padi % N]).reshape(PAD_E // K, K)
    dstp = jnp.concatenate([dst, N + (padi % N_SINK)]).reshape(PAD_E // K, K)
    ones_k = jnp.ones((K, IN_DIM), jnp.float32)
    z128 = jnp.zeros((NP, IN_DIM), jnp.float32)
    w3p = jnp.pad(W3, ((0, 0), (0, IN_DIM - NCLS)))

    degp = _make_deg()(ones_k, dstp, z128)
    dis, x0 = _tc_prep(degp, X)
    y1p = _make_agg(128, False)(x0, srcp, dstp, z128)
    t2s = _tc_layer1(y1p, x0, dis, W1, b1.reshape(1, HID), W2)
    y2s = _make_agg(128, True)(t2s, srcp, dstp, z128)
    t3 = _tc_layer2(y2s, t2s, dis, b2.reshape(1, HID), w3p)
    y3p = _make_agg(128, False)(t3, srcp, dstp, z128)
    return _tc_final(y3p, t3, dis, b3.reshape(1, NCLS))


# speedup vs baseline: 1.1287x; 1.0022x over previous
import functools

import jax
import jax.numpy as jnp
from jax import lax
from jax.experimental import pallas as pl
from jax.experimental.pallas import tpu as pltpu
from jax.experimental.pallas import tpu_sc as plsc

N = 10000
E = 320000
IN_DIM = 128
HID = 256
NCLS = 64

NC, NS = 2, 16
NW = NC * NS
K = 80
PAD_E = 327680
NSLOT = 4
N_SINK = 16
NP = N + N_SINK
TPT = 624
TAIL_I = NP - TPT * NS
TAIL_W = N - TPT * NS


@functools.lru_cache(maxsize=None)
def _make_agg(dc, feat_split):
    epw = PAD_E // NS if feat_split else PAD_E // NW
    nch = epw // K
    SCH = NSLOT
    nsc = nch // SCH
    mesh = plsc.VectorSubcoreMesh(core_axis_name="c", subcore_axis_name="s",
                                  num_cores=NC, num_subcores=NS)

    def body(x_hbm, src_hbm, dst_hbm, zero_hbm, out_hbm,
             sidx, didx, rows, acc, gsem0, gsem1, gsem2, gsem3,
             ssem0, ssem1, ssem2, ssem3, isem, zsem):
        gsem = {0: gsem0, 1: gsem1, 2: gsem2, 3: gsem3}
        ssem = {0: ssem0, 1: ssem1, 2: ssem2, 3: ssem3}
        c = lax.axis_index("c")
        s = lax.axis_index("s")
        crow = (s if feat_split else c * NS + s) * nch
        xref = x_hbm.at[c] if feat_split else x_hbm

        def idx_load(scn, p):
            r = crow + scn * SCH
            pltpu.async_copy(src_hbm.at[pl.ds(r, SCH)], sidx.at[p], isem)
            pltpu.async_copy(dst_hbm.at[pl.ds(r, SCH)], didx.at[p], isem)

        def idx_wait(scn, p):
            r = crow + scn * SCH
            pltpu.make_async_copy(src_hbm.at[pl.ds(r, SCH)], sidx.at[p],
                                  isem).wait()
            pltpu.make_async_copy(dst_hbm.at[pl.ds(r, SCH)], didx.at[p],
                                  isem).wait()

        def start_g(p, k, b):
            pltpu.async_copy(xref.at[sidx.at[p, k]], rows.at[b], gsem[b])

        def wait_g(p, k, b):
            pltpu.make_async_copy(xref.at[sidx.at[p, k]], rows.at[b],
                                  gsem[b]).wait()

        def start_s(p, k, b):
            pltpu.async_copy(rows.at[b], acc.at[didx.at[p, k]],
                             ssem[b], add=True)

        def wait_s(p, k, b):
            pltpu.make_async_copy(rows.at[b], acc.at[didx.at[p, k]],
                                  ssem[b]).wait()

        ini = pltpu.async_copy(zero_hbm.at[pl.ds(s * TPT, TPT)],
                               acc.at[pl.ds(s * TPT, TPT)], zsem)

        @pl.when(s == 0)
        def _init_tail():
            pltpu.async_copy(zero_hbm.at[pl.ds(TPT * NS, TAIL_I)],
                             acc.at[pl.ds(TPT * NS, TAIL_I)], zsem)

        pltpu.sync_copy(src_hbm.at[pl.ds(crow, SCH)], sidx.at[0])
        pltpu.sync_copy(dst_hbm.at[pl.ds(crow, SCH)], didx.at[0])
        idx_load(1, 1)
        ini.wait()

        @pl.when(s == 0)
        def _init_tail_wait():
            pltpu.make_async_copy(zero_hbm.at[pl.ds(TPT * NS, TAIL_I)],
                                  acc.at[pl.ds(TPT * NS, TAIL_I)],
                                  zsem).wait()

        plsc.subcore_barrier()

        start_g(0, 0, 0)
        start_g(0, 1, 1)
        start_g(0, 2, 2)

        def sc_body(scn, carry):
            p = scn % 2
            for k in range(SCH):
                wait_g(p, k, k)
                start_s(p, k, k)
                if k > 0:
                    wait_s(p, k - 1, k - 1)
                if k == 0:
                    start_g(p, 3, 3)
                elif k == 1:
                    @pl.when(scn + 1 < nsc)
                    def _pf0():
                        idx_wait(scn + 1, 1 - p)
                        start_g(1 - p, 0, 0)
                elif k == 2:
                    @pl.when(scn + 1 < nsc)
                    def _pf1():
                        start_g(1 - p, 1, 1)
                else:
                    @pl.when(scn + 1 < nsc)
                    def _pf2():
                        start_g(1 - p, 2, 2)
            wait_s(p, SCH - 1, SCH - 1)
            @pl.when(scn + 2 < nsc)
            def _pfi():
                idx_load(scn + 2, p)
            return carry

        lax.fori_loop(0, nsc, sc_body, 0)

        plsc.subcore_barrier()
        pltpu.sync_copy(acc.at[pl.ds(s * TPT, TPT)],
                        out_hbm.at[c, pl.ds(s * TPT, TPT)])

        @pl.when(s == 0)
        def _write_tail():
            pltpu.sync_copy(acc.at[pl.ds(TPT * NS, TAIL_W)],
                            out_hbm.at[c, pl.ds(TPT * NS, TAIL_W)])

    return pl.kernel(
        body,
        out_type=jax.ShapeDtypeStruct((NC, N, dc), jnp.float32),
        mesh=mesh,
        scratch_types=[
            pltpu.VMEM((2, SCH, K), jnp.int32),
            pltpu.VMEM((2, SCH, K), jnp.int32),
            pltpu.VMEM((NSLOT, K, dc), jnp.float32),
            pltpu.VMEM_SHARED((NP, dc), jnp.float32),
            pltpu.SemaphoreType.DMA,
            pltpu.SemaphoreType.DMA,
            pltpu.SemaphoreType.DMA,
            pltpu.SemaphoreType.DMA,
            pltpu.SemaphoreType.DMA,
            pltpu.SemaphoreType.DMA,
            pltpu.SemaphoreType.DMA,
            pltpu.SemaphoreType.DMA,
            pltpu.SemaphoreType.DMA,
            pltpu.SemaphoreType.DMA,
        ],
    )


@functools.lru_cache(maxsize=None)
def _make_deg():
    nch = PAD_E // NW // K
    SCH = 8
    nsc = nch // SCH
    mesh = plsc.VectorSubcoreMesh(core_axis_name="c", subcore_axis_name="s",
                                  num_cores=NC, num_subcores=NS)

    def body(ones_hbm, dst_hbm, zero_hbm, out_hbm,
             didx, rows, acc, ssem0, ssem1, ssem2, ssem3, isem, zsem):
        ssem = {0: ssem0, 1: ssem1, 2: ssem2, 3: ssem3}
        c = lax.axis_index("c")
        s = lax.axis_index("s")
        crow = (c * NS + s) * nch

        def idx_load(scn, p):
            pltpu.async_copy(dst_hbm.at[pl.ds(crow + scn * SCH, SCH)],
                             didx.at[p], isem)

        def idx_wait(scn, p):
            pltpu.make_async_copy(dst_hbm.at[pl.ds(crow + scn * SCH, SCH)],
                                  didx.at[p], isem).wait()

        def start_s(p, k, b):
            pltpu.async_copy(rows, acc.at[didx.at[p, k]], ssem[b % 4],
                             add=True)

        def wait_s(p, k, b):
            pltpu.make_async_copy(rows, acc.at[didx.at[p, k]],
                                  ssem[b % 4]).wait()

        ini = pltpu.async_copy(zero_hbm.at[pl.ds(s * TPT, TPT)],
                               acc.at[pl.ds(s * TPT, TPT)], zsem)

        @pl.when(s == 0)
        def _init_tail():
            pltpu.async_copy(zero_hbm.at[pl.ds(TPT * NS, TAIL_I)],
                             acc.at[pl.ds(TPT * NS, TAIL_I)], zsem)

        pltpu.sync_copy(dst_hbm.at[pl.ds(crow, SCH)], didx.at[0])
        idx_load(1, 1)
        pltpu.sync_copy(ones_hbm, rows)
        ini.wait()

        @pl.when(s == 0)
        def _init_tail_wait():
            pltpu.make_async_copy(zero_hbm.at[pl.ds(TPT * NS, TAIL_I)],
                                  acc.at[pl.ds(TPT * NS, TAIL_I)],
                                  zsem).wait()

        plsc.subcore_barrier()

        def sc_body(scn, carry):
            p = scn % 2
            for k in range(SCH):
                start_s(p, k, k)
                if k >= 3:
                    wait_s(p, k - 3, k - 3)
                if k == SCH - 2:
                    @pl.when(scn + 1 < nsc)
                    def _pf():
                        idx_wait(scn + 1, 1 - p)
            for k in (SCH - 3, SCH - 2, SCH - 1):
                wait_s(p, k, k)
            @pl.when(scn + 2 < nsc)
            def _pfi():
                idx_load(scn + 2, p)
            return carry

        lax.fori_loop(0, nsc, sc_body, 0)

        plsc.subcore_barrier()
        pltpu.sync_copy(acc.at[pl.ds(s * TPT, TPT)],
                        out_hbm.at[c, pl.ds(s * TPT, TPT)])

        @pl.when(s == 0)
        def _write_tail():
            pltpu.sync_copy(acc.at[pl.ds(TPT * NS, TAIL_W)],
                            out_hbm.at[c, pl.ds(TPT * NS, TAIL_W)])

    return pl.kernel(
        body,
        out_type=jax.ShapeDtypeStruct((NC, N, IN_DIM), jnp.float32),
        mesh=mesh,
        scratch_types=[
            pltpu.VMEM((2, SCH, K), jnp.int32),
            pltpu.VMEM((K, IN_DIM), jnp.float32),
            pltpu.VMEM_SHARED((NP, IN_DIM), jnp.float32),
            pltpu.SemaphoreType.DMA,
            pltpu.SemaphoreType.DMA,
            pltpu.SemaphoreType.DMA,
            pltpu.SemaphoreType.DMA,
            pltpu.SemaphoreType.DMA,
            pltpu.SemaphoreType.DMA,
        ],
    )


_RB = 1000


def _tc_prep(degp, x):
    def body(dp, xr, dis_o, x0_o):
        deg = dp[0, :, 0:1] + dp[1, :, 0:1] + 1.0
        d = lax.rsqrt(deg)
        dis_o[...] = d
        x0_o[...] = xr[...] * d

    return pl.pallas_call(
        body,
        grid=(N // _RB,),
        in_specs=[pl.BlockSpec((NC, _RB, IN_DIM), lambda i: (0, i, 0)),
                  pl.BlockSpec((_RB, IN_DIM), lambda i: (i, 0))],
        out_specs=[pl.BlockSpec((_RB, 1), lambda i: (i, 0)),
                   pl.BlockSpec((_RB, IN_DIM), lambda i: (i, 0))],
        out_shape=[jax.ShapeDtypeStruct((N, 1), jnp.float32),
                   jax.ShapeDtypeStruct((N, IN_DIM), jnp.float32)],
    )(degp, x)


def _tc_layer1(y1p, x0, dis, w1, b1, w2):
    def body(yp, x0r, dr, w1r, b1r, w2r, out):
        z = (yp[0] + yp[1] + x0r[...]) * dr[...]
        h = jnp.maximum(
            jnp.dot(z, w1r[...], preferred_element_type=jnp.float32) + b1r[...],
            0.0)
        t = jnp.dot(h, w2r[...], preferred_element_type=jnp.float32) * dr[...]
        out[0, :, :] = t[:, :IN_DIM]
        out[1, :, :] = t[:, IN_DIM:]

    return pl.pallas_call(
        body,
        grid=(N // _RB,),
        in_specs=[pl.BlockSpec((NC, _RB, IN_DIM), lambda i: (0, i, 0)),
                  pl.BlockSpec((_RB, IN_DIM), lambda i: (i, 0)),
                  pl.BlockSpec((_RB, 1), lambda i: (i, 0)),
                  pl.BlockSpec((IN_DIM, HID), lambda i: (0, 0)),
                  pl.BlockSpec((1, HID), lambda i: (0, 0)),
                  pl.BlockSpec((HID, HID), lambda i: (0, 0))],
        out_specs=pl.BlockSpec((NC, _RB, IN_DIM), lambda i: (0, i, 0)),
        out_shape=jax.ShapeDtypeStruct((NC, N, IN_DIM), jnp.float32),
    )(y1p, x0, dis, w1, b1, w2)


def _tc_layer2(y2s, t2s, dis, b2, w3):
    def body(yr, tr, dr, b2r, w3r, out):
        u0 = (yr[0] + tr[0]) * dr[...]
        u1 = (yr[1] + tr[1]) * dr[...]
        h = jnp.maximum(jnp.concatenate([u0, u1], axis=1) + b2r[...], 0.0)
        out[...] = jnp.dot(h, w3r[...],
                           preferred_element_type=jnp.float32) * dr[...]

    return pl.pallas_call(
        body,
        grid=(N // _RB,),
        in_specs=[pl.BlockSpec((NC, _RB, IN_DIM), lambda i: (0, i, 0)),
                  pl.BlockSpec((NC, _RB, IN_DIM), lambda i: (0, i, 0)),
                  pl.BlockSpec((_RB, 1), lambda i: (i, 0)),
                  pl.BlockSpec((1, HID), lambda i: (0, 0)),
                  pl.BlockSpec((HID, IN_DIM), lambda i: (0, 0))],
        out_specs=pl.BlockSpec((_RB, IN_DIM), lambda i: (i, 0)),
        out_shape=jax.ShapeDtypeStruct((N, IN_DIM), jnp.float32),
    )(y2s, t2s, dis, b2, w3)


def _tc_final(y3p, t3, dis, b3):
    def body(yp, tr, dr, b3r, out):
        out[...] = ((yp[0] + yp[1] + tr[...]) * dr[...])[:, :NCLS] + b3r[...]

    return pl.pallas_call(
        body,
        grid=(N // _RB,),
        in_specs=[pl.BlockSpec((NC, _RB, IN_DIM), lambda i: (0, i, 0)),
                  pl.BlockSpec((_RB, IN_DIM), lambda i: (i, 0)),
                  pl.BlockSpec((_RB, 1), lambda i: (i, 0)),
                  pl.BlockSpec((1, NCLS), lambda i: (0, 0))],
        out_specs=pl.BlockSpec((_RB, NCLS), lambda i: (i, 0)),
        out_shape=jax.ShapeDtypeStruct((N, NCLS), jnp.float32),
    )(y3p, t3, dis, b3)


def kernel(X, A, W1, b1, W2, b2, W3, b3):
    src = A[0].astype(jnp.int32)
    dst = A[1].astype(jnp.int32)
    padi = jnp.arange(PAD_E - E, dtype=jnp.int32)
    srcp = jnp.concatenate([src, padi % N]).reshape(PAD_E // K, K)
    dstp = jnp.concatenate([dst, N + (padi % N_SINK)]).reshape(PAD_E // K, K)
    ones_k = jnp.ones((K, IN_DIM), jnp.float32)
    z128 = jnp.zeros((NP, IN_DIM), jnp.float32)
    w3p = jnp.pad(W3, ((0, 0), (0, IN_DIM - NCLS)))

    degp = _make_deg()(ones_k, dstp, z128)
    dis, x0 = _tc_prep(degp, X)
    y1p = _make_agg(128, False)(x0, srcp, dstp, z128)
    t2s = _tc_layer1(y1p, x0, dis, W1, b1.reshape(1, HID), W2)
    y2s = _make_agg(128, True)(t2s, srcp, dstp, z128)
    t3 = _tc_layer2(y2s, t2s, dis, b2.reshape(1, HID), w3p)
    y3p = _make_agg(128, False)(t3, srcp, dstp, z128)
    return _tc_final(y3p, t3, dis, b3.reshape(1, NCLS))
